# baseline (speedup 1.0000x reference)
import functools

import jax
import jax.numpy as jnp
from jax import lax
from jax.experimental import pallas as pl
from jax.experimental.pallas import tpu as pltpu
from jax.experimental.pallas import tpu_sc as plsc

N = 10000
E = 160000
HID = 512
LANE = 128

NUM_SC = 2
NUM_TILES = 16
EPT = E // NUM_TILES
EB = 128
SLACK = 1080
CAP = EPT + 2 * SLACK
RMAX = 24
CAP2 = CAP + EB * RMAX
NB = CAP2 // EB
NJUNK = 8
ROWS_PT = 632
ROWS_LAST = N + NJUNK - (NUM_TILES - 1) * ROWS_PT


@functools.partial(jax.jit, static_argnames=("nchunks",))
def _sc_segsum(h_ch, src_main, dst_main, zeros, *, nchunks):
    cpc = nchunks // NUM_SC
    mesh = plsc.VectorSubcoreMesh(core_axis_name="c", subcore_axis_name="s",
                                  num_cores=NUM_SC, num_subcores=NUM_TILES)

    @functools.partial(
        pl.kernel,
        out_type=jax.ShapeDtypeStruct((nchunks, N, LANE), jnp.float32),
        mesh=mesh,
        scratch_types=[
            pltpu.VMEM((NB, EB), jnp.int32),
            pltpu.VMEM((NB, EB), jnp.int32),
            pltpu.VMEM((EB, LANE), jnp.float32),
            pltpu.VMEM_SHARED((N + NJUNK, LANE), jnp.float32),
            pltpu.SemaphoreType.DMA,
        ],
    )
    def seg(h_hbm, srcm_hbm, dstm_hbm, zeros_hbm, out_hbm,
            src_v, dst_v, rows_a, acc, sem_a):
        sid = lax.axis_index("s")
        cid = lax.axis_index("c")
        pltpu.sync_copy(srcm_hbm.at[sid], src_v)
        pltpu.sync_copy(dstm_hbm.at[sid], dst_v)

        def run_core(core):
            def _go():
                off = pl.multiple_of(sid * ROWS_PT, 8)
                for j in range(cpc):
                    chunk = core * cpc + j
                    hc = h_hbm.at[chunk]
                    @pl.when(sid < NUM_TILES - 1)
                    def _():
                        pltpu.sync_copy(zeros_hbm, acc.at[pl.ds(off, ROWS_PT)])

                    @pl.when(sid == NUM_TILES - 1)
                    def _():
                        pltpu.sync_copy(zeros_hbm.at[pl.ds(0, ROWS_LAST)],
                                        acc.at[pl.ds((NUM_TILES - 1) * ROWS_PT,
                                                     ROWS_LAST)])
                    plsc.subcore_barrier()

                    def body(it, _):
                        pltpu.async_copy(hc.at[src_v.at[it]], rows_a, sem_a).wait()
                        pltpu.sync_copy(rows_a, acc.at[dst_v.at[it]], add=True)
                        return 0

                    lax.fori_loop(0, NB, body, 0)
                    plsc.subcore_barrier()

                    @pl.when(sid < NUM_TILES - 1)
                    def _():
                        pltpu.sync_copy(acc.at[pl.ds(off, ROWS_PT)],
                                        out_hbm.at[chunk].at[pl.ds(off, ROWS_PT)])

                    @pl.when(sid == NUM_TILES - 1)
                    def _():
                        base = (NUM_TILES - 1) * ROWS_PT
                        nlast = N - base
                        pltpu.sync_copy(acc.at[pl.ds(base, nlast)],
                                        out_hbm.at[chunk].at[pl.ds(base, nlast)])
            return _go

        for core in range(NUM_SC):
            pl.when(cid == core)(run_core(core))

    return seg(h_ch, src_main, dst_main, zeros)


BN_ROWS = 400
GRID = N // BN_ROWS


def _mlp_body(h_ref, a_ref, eps_ref, w1_ref, b1_ref, w2_ref, b2_ref,
              z_ref, s_ref, *, cin):
    i = pl.program_id(0)
    epsv = 1.0 + eps_ref[0, 0]
    z = jnp.concatenate([epsv * h_ref[c] + a_ref[c] for c in range(cin)], axis=1)
    z1 = jnp.dot(z, w1_ref[...], preferred_element_type=jnp.float32) + b1_ref[...]
    z1 = jnp.maximum(z1, 0.0)
    z2 = jnp.dot(z1, w2_ref[...], preferred_element_type=jnp.float32) + b2_ref[...]
    for k in range(HID // LANE):
        z_ref[k] = z2[:, k * LANE:(k + 1) * LANE]
    ps = jnp.concatenate([jnp.sum(z2, axis=0, keepdims=True),
                          jnp.sum(z2 * z2, axis=0, keepdims=True)], axis=0)

    @pl.when(i == 0)
    def _():
        s_ref[...] = ps

    @pl.when(i > 0)
    def _():
        s_ref[...] = s_ref[...] + ps


def _tc_mlp(h_ch, agg_ch, eps, w1, b1, w2, b2):
    cin = h_ch.shape[0]
    co = HID // LANE
    return pl.pallas_call(
        functools.partial(_mlp_body, cin=cin),
        grid=(GRID,),
        in_specs=[
            pl.BlockSpec((cin, BN_ROWS, LANE), lambda i: (0, i, 0)),
            pl.BlockSpec((cin, BN_ROWS, LANE), lambda i: (0, i, 0)),
            pl.BlockSpec(memory_space=pltpu.SMEM),
            pl.BlockSpec((cin * LANE, HID), lambda i: (0, 0)),
            pl.BlockSpec((1, HID), lambda i: (0, 0)),
            pl.BlockSpec((HID, HID), lambda i: (0, 0)),
            pl.BlockSpec((1, HID), lambda i: (0, 0)),
        ],
        out_specs=[
            pl.BlockSpec((co, BN_ROWS, LANE), lambda i: (0, i, 0)),
            pl.BlockSpec((2, HID), lambda i: (0, 0)),
        ],
        out_shape=[
            jax.ShapeDtypeStruct((co, N, LANE), jnp.float32),
            jax.ShapeDtypeStruct((2, HID), jnp.float32),
        ],
    )(h_ch, agg_ch, eps, w1, b1, w2, b2)


def _norm_body(z_ref, sc_ref, sh_ref, h_ref):
    for k in range(HID // LANE):
        h_ref[k] = jnp.maximum(z_ref[k] * sc_ref[k] + sh_ref[k], 0.0)


def _tc_norm(z_ch, scale4, shift4):
    co = HID // LANE
    return pl.pallas_call(
        _norm_body,
        grid=(GRID,),
        in_specs=[
            pl.BlockSpec((co, BN_ROWS, LANE), lambda i: (0, i, 0)),
            pl.BlockSpec((co, 1, LANE), lambda i: (0, 0, 0)),
            pl.BlockSpec((co, 1, LANE), lambda i: (0, 0, 0)),
        ],
        out_specs=pl.BlockSpec((co, BN_ROWS, LANE), lambda i: (0, i, 0)),
        out_shape=jax.ShapeDtypeStruct((co, N, LANE), jnp.float32),
    )(z_ch, scale4, shift4)


def _heads_body(h_ref, wo_ref, bo_ref, wm_ref, bm_ref, wl_ref, bl_ref,
                mu_ref, lv_ref):
    h = jnp.concatenate([h_ref[k] for k in range(HID // LANE)], axis=1)
    ne = jnp.dot(h, wo_ref[...], preferred_element_type=jnp.float32) + bo_ref[...]
    mu_ref[...] = jnp.dot(ne, wm_ref[...], preferred_element_type=jnp.float32) + bm_ref[...]
    lv_ref[...] = jnp.dot(ne, wl_ref[...], preferred_element_type=jnp.float32) + bl_ref[...]


def _tc_heads(h_ch, wo, bo, wm, bm, wl, bl):
    co = HID // LANE
    dv = wm.shape[1]
    return pl.pallas_call(
        _heads_body,
        grid=(GRID,),
        in_specs=[
            pl.BlockSpec((co, BN_ROWS, LANE), lambda i: (0, i, 0)),
            pl.BlockSpec((HID, HID), lambda i: (0, 0)),
            pl.BlockSpec((1, HID), lambda i: (0, 0)),
            pl.BlockSpec((HID, dv), lambda i: (0, 0)),
            pl.BlockSpec((1, dv), lambda i: (0, 0)),
            pl.BlockSpec((HID, dv), lambda i: (0, 0)),
            pl.BlockSpec((1, dv), lambda i: (0, 0)),
        ],
        out_specs=[
            pl.BlockSpec((BN_ROWS, dv), lambda i: (i, 0)),
            pl.BlockSpec((BN_ROWS, dv), lambda i: (i, 0)),
        ],
        out_shape=[
            jax.ShapeDtypeStruct((N, dv), jnp.float32),
            jax.ShapeDtypeStruct((N, dv), jnp.float32),
        ],
    )(h_ch, wo, bo, wm, bm, wl, bl)


def kernel(x, edge_index, params):
    order = jnp.argsort(edge_index[1], stable=True)
    src = edge_index[0][order]
    dst = edge_index[1][order]
    targets = jnp.arange(1, NUM_TILES) * EPT
    raw = jnp.searchsorted(dst, dst[targets], side='left')
    bnd = jnp.clip(raw, targets - SLACK, targets + SLACK)
    bnd = jnp.concatenate([jnp.zeros((1,), bnd.dtype), bnd,
                           jnp.full((1,), E, bnd.dtype)])
    pos = bnd[:NUM_TILES, None] + jnp.arange(CAP)[None, :]
    valid = pos < bnd[1:, None]
    posc = jnp.minimum(pos, E - 1)
    src_t = jnp.where(valid, src[posc], 0).astype(jnp.int32)
    dst_t = jnp.where(valid, dst[posc], N).astype(jnp.int32)

    def _wave(dst_row, src_row):
        run_start = jnp.searchsorted(dst_row, dst_row, side='left')
        rank = jnp.minimum(jnp.arange(CAP, dtype=jnp.int32) - run_start, RMAX)
        sort_ids = jnp.argsort(rank, stable=True)
        ranks_sorted = rank[sort_ids]
        w = jnp.sum(ranks_sorted[None, :] == jnp.arange(RMAX + 1)[:, None],
                    axis=1)
        wpad = jnp.where(jnp.arange(RMAX + 1) < RMAX,
                         ((w + EB - 1) // EB) * EB, w)
        offs = jnp.concatenate([jnp.zeros((1,), jnp.int32),
                                jnp.cumsum(wpad)[:-1].astype(jnp.int32)])
        in_wave = (jnp.arange(CAP, dtype=jnp.int32)
                   - jnp.searchsorted(ranks_sorted, ranks_sorted, side='left'
                                      ).astype(jnp.int32))
        newpos = offs[ranks_sorted] + in_wave
        out_dst = jnp.full((CAP2,), N, jnp.int32).at[newpos].set(dst_row[sort_ids])
        out_src = jnp.zeros((CAP2,), jnp.int32).at[newpos].set(src_row[sort_ids])
        return out_src, out_dst

    src_main, dst_main = jax.vmap(_wave)(dst_t, src_t)
    src_main = src_main.reshape(NUM_TILES, NB, EB)
    dst_main = dst_main.reshape(NUM_TILES, NB, EB)
    zeros = jnp.zeros((ROWS_PT, LANE), jnp.float32)

    h_ch = x.reshape(N, 2, LANE).transpose(1, 0, 2)
    nlayers = 4
    for l in range(nlayers):
        cin = h_ch.shape[0]
        agg_ch = _sc_segsum(h_ch, src_main, dst_main, zeros, nchunks=cin)
        eps = params[f'eps_{l}'].reshape(1, 1)
        b1 = params[f'b1_{l}'].reshape(1, HID)
        b2 = params[f'b2_{l}'].reshape(1, HID)
        z_ch, sums = _tc_mlp(h_ch, agg_ch, eps, params[f'W1_{l}'], b1,
                             params[f'W2_{l}'], b2)
        mean = sums[0] / N
        var = sums[1] / N - mean * mean
        inv = lax.rsqrt(var + 1e-5)
        scale = params[f'gamma_{l}'] * inv
        shift = params[f'beta_{l}'] - mean * scale
        h_ch = _tc_norm(z_ch, scale.reshape(HID // LANE, 1, LANE),
                        shift.reshape(HID // LANE, 1, LANE))

    bo = params['b_out'].reshape(1, HID)
    bm = params['b_mu'].reshape(1, -1)
    bl = params['b_lv'].reshape(1, -1)
    return _tc_heads(h_ch, params['W_out'], bo, params['W_mu'], bm,
                     params['W_lv'], bl)


# speedup vs baseline: 1.0009x; 1.0009x over previous
import functools

import jax
import jax.numpy as jnp
from jax import lax
from jax.experimental import pallas as pl
from jax.experimental.pallas import tpu as pltpu
from jax.experimental.pallas import tpu_sc as plsc

N = 10000
E = 160000
HID = 512
LANE = 128

NUM_SC = 2
NUM_TILES = 16
EPT = E // NUM_TILES
EB = 128
SLACK = 1080
CAP = EPT + 2 * SLACK
RMAX = 24
CAP2 = CAP + EB * RMAX
NB = CAP2 // EB
GLEN = 17
GROUPS = NB // GLEN
NJUNK = 8
ROWS_PT = 632
ROWS_LAST = N + NJUNK - (NUM_TILES - 1) * ROWS_PT


@functools.partial(jax.jit, static_argnames=("nchunks",))
def _sc_segsum(h_ch, src_main, dst_main, zeros, *, nchunks):
    cpc = nchunks // NUM_SC
    mesh = plsc.VectorSubcoreMesh(core_axis_name="c", subcore_axis_name="s",
                                  num_cores=NUM_SC, num_subcores=NUM_TILES)

    @functools.partial(
        pl.kernel,
        out_type=jax.ShapeDtypeStruct((nchunks, N, LANE), jnp.float32),
        mesh=mesh,
        scratch_types=[
            pltpu.VMEM((GLEN, EB), jnp.int32),
            pltpu.VMEM((GLEN, EB), jnp.int32),
            pltpu.VMEM((EB, LANE), jnp.float32),
            pltpu.VMEM((EB, LANE), jnp.float32),
            pltpu.VMEM_SHARED((N + NJUNK, LANE), jnp.float32),
            pltpu.SemaphoreType.DMA,
            pltpu.SemaphoreType.DMA,
        ],
    )
    def seg(h_hbm, srcm_hbm, dstm_hbm, zeros_hbm, out_hbm,
            src_v, dst_v, rows_a, rows_b, acc, sem_a, sem_b):
        sid = lax.axis_index("s")
        cid = lax.axis_index("c")
        rows = (rows_a, rows_b)
        sems = (sem_a, sem_b)

        def run_core(core):
            def _go():
                off = pl.multiple_of(sid * ROWS_PT, 8)
                for j in range(cpc):
                    chunk = core * cpc + j
                    hc = h_hbm.at[chunk]
                    @pl.when(sid < NUM_TILES - 1)
                    def _():
                        pltpu.sync_copy(zeros_hbm, acc.at[pl.ds(off, ROWS_PT)])

                    @pl.when(sid == NUM_TILES - 1)
                    def _():
                        pltpu.sync_copy(zeros_hbm.at[pl.ds(0, ROWS_LAST)],
                                        acc.at[pl.ds((NUM_TILES - 1) * ROWS_PT,
                                                     ROWS_LAST)])
                    plsc.subcore_barrier()

                    def gbody(g, _):
                        pltpu.sync_copy(srcm_hbm.at[sid].at[g], src_v)
                        pltpu.sync_copy(dstm_hbm.at[sid].at[g], dst_v)
                        pltpu.async_copy(hc.at[src_v.at[0]], rows[0], sems[0])
                        for b in range(GLEN):
                            pltpu.make_async_copy(hc.at[src_v.at[b]],
                                                  rows[b % 2], sems[b % 2]).wait()
                            if b + 1 < GLEN:
                                pltpu.async_copy(hc.at[src_v.at[b + 1]],
                                                 rows[(b + 1) % 2],
                                                 sems[(b + 1) % 2])
                            pltpu.sync_copy(rows[b % 2],
                                            acc.at[dst_v.at[b]], add=True)
                        return 0

                    lax.fori_loop(0, GROUPS, gbody, 0)
                    plsc.subcore_barrier()

                    @pl.when(sid < NUM_TILES - 1)
                    def _():
                        pltpu.sync_copy(acc.at[pl.ds(off, ROWS_PT)],
                                        out_hbm.at[chunk].at[pl.ds(off, ROWS_PT)])

                    @pl.when(sid == NUM_TILES - 1)
                    def _():
                        base = (NUM_TILES - 1) * ROWS_PT
                        nlast = N - base
                        pltpu.sync_copy(acc.at[pl.ds(base, nlast)],
                                        out_hbm.at[chunk].at[pl.ds(base, nlast)])
            return _go

        for core in range(NUM_SC):
            pl.when(cid == core)(run_core(core))

    return seg(h_ch, src_main, dst_main, zeros)


BN_ROWS = 400
GRID = N // BN_ROWS


def _mlp_body(h_ref, a_ref, eps_ref, w1_ref, b1_ref, w2_ref, b2_ref,
              z_ref, s_ref, *, cin):
    i = pl.program_id(0)
    epsv = 1.0 + eps_ref[0, 0]
    z = jnp.concatenate([epsv * h_ref[c] + a_ref[c] for c in range(cin)], axis=1)
    z1 = jnp.dot(z, w1_ref[...], preferred_element_type=jnp.float32) + b1_ref[...]
    z1 = jnp.maximum(z1, 0.0)
    z2 = jnp.dot(z1, w2_ref[...], preferred_element_type=jnp.float32) + b2_ref[...]
    for k in range(HID // LANE):
        z_ref[k] = z2[:, k * LANE:(k + 1) * LANE]
    ps = jnp.concatenate([jnp.sum(z2, axis=0, keepdims=True),
                          jnp.sum(z2 * z2, axis=0, keepdims=True)], axis=0)

    @pl.when(i == 0)
    def _():
        s_ref[...] = ps

    @pl.when(i > 0)
    def _():
        s_ref[...] = s_ref[...] + ps


def _tc_mlp(h_ch, agg_ch, eps, w1, b1, w2, b2):
    cin = h_ch.shape[0]
    co = HID // LANE
    return pl.pallas_call(
        functools.partial(_mlp_body, cin=cin),
        grid=(GRID,),
        in_specs=[
            pl.BlockSpec((cin, BN_ROWS, LANE), lambda i: (0, i, 0)),
            pl.BlockSpec((cin, BN_ROWS, LANE), lambda i: (0, i, 0)),
            pl.BlockSpec(memory_space=pltpu.SMEM),
            pl.BlockSpec((cin * LANE, HID), lambda i: (0, 0)),
            pl.BlockSpec((1, HID), lambda i: (0, 0)),
            pl.BlockSpec((HID, HID), lambda i: (0, 0)),
            pl.BlockSpec((1, HID), lambda i: (0, 0)),
        ],
        out_specs=[
            pl.BlockSpec((co, BN_ROWS, LANE), lambda i: (0, i, 0)),
            pl.BlockSpec((2, HID), lambda i: (0, 0)),
        ],
        out_shape=[
            jax.ShapeDtypeStruct((co, N, LANE), jnp.float32),
            jax.ShapeDtypeStruct((2, HID), jnp.float32),
        ],
    )(h_ch, agg_ch, eps, w1, b1, w2, b2)


def _norm_body(z_ref, sc_ref, sh_ref, h_ref):
    for k in range(HID // LANE):
        h_ref[k] = jnp.maximum(z_ref[k] * sc_ref[k] + sh_ref[k], 0.0)


def _tc_norm(z_ch, scale4, shift4):
    co = HID // LANE
    return pl.pallas_call(
        _norm_body,
        grid=(GRID,),
        in_specs=[
            pl.BlockSpec((co, BN_ROWS, LANE), lambda i: (0, i, 0)),
            pl.BlockSpec((co, 1, LANE), lambda i: (0, 0, 0)),
            pl.BlockSpec((co, 1, LANE), lambda i: (0, 0, 0)),
        ],
        out_specs=pl.BlockSpec((co, BN_ROWS, LANE), lambda i: (0, i, 0)),
        out_shape=jax.ShapeDtypeStruct((co, N, LANE), jnp.float32),
    )(z_ch, scale4, shift4)


def _heads_body(h_ref, wo_ref, bo_ref, wm_ref, bm_ref, wl_ref, bl_ref,
                mu_ref, lv_ref):
    h = jnp.concatenate([h_ref[k] for k in range(HID // LANE)], axis=1)
    ne = jnp.dot(h, wo_ref[...], preferred_element_type=jnp.float32) + bo_ref[...]
    mu_ref[...] = jnp.dot(ne, wm_ref[...], preferred_element_type=jnp.float32) + bm_ref[...]
    lv_ref[...] = jnp.dot(ne, wl_ref[...], preferred_element_type=jnp.float32) + bl_ref[...]


def _tc_heads(h_ch, wo, bo, wm, bm, wl, bl):
    co = HID // LANE
    dv = wm.shape[1]
    return pl.pallas_call(
        _heads_body,
        grid=(GRID,),
        in_specs=[
            pl.BlockSpec((co, BN_ROWS, LANE), lambda i: (0, i, 0)),
            pl.BlockSpec((HID, HID), lambda i: (0, 0)),
            pl.BlockSpec((1, HID), lambda i: (0, 0)),
            pl.BlockSpec((HID, dv), lambda i: (0, 0)),
            pl.BlockSpec((1, dv), lambda i: (0, 0)),
            pl.BlockSpec((HID, dv), lambda i: (0, 0)),
            pl.BlockSpec((1, dv), lambda i: (0, 0)),
        ],
        out_specs=[
            pl.BlockSpec((BN_ROWS, dv), lambda i: (i, 0)),
            pl.BlockSpec((BN_ROWS, dv), lambda i: (i, 0)),
        ],
        out_shape=[
            jax.ShapeDtypeStruct((N, dv), jnp.float32),
            jax.ShapeDtypeStruct((N, dv), jnp.float32),
        ],
    )(h_ch, wo, bo, wm, bm, wl, bl)


def kernel(x, edge_index, params):
    order = jnp.argsort(edge_index[1], stable=True)
    src = edge_index[0][order]
    dst = edge_index[1][order]
    targets = jnp.arange(1, NUM_TILES) * EPT
    raw = jnp.searchsorted(dst, dst[targets], side='left')
    bnd = jnp.clip(raw, targets - SLACK, targets + SLACK)
    bnd = jnp.concatenate([jnp.zeros((1,), bnd.dtype), bnd,
                           jnp.full((1,), E, bnd.dtype)])
    pos = bnd[:NUM_TILES, None] + jnp.arange(CAP)[None, :]
    valid = pos < bnd[1:, None]
    posc = jnp.minimum(pos, E - 1)
    src_t = jnp.where(valid, src[posc], 0).astype(jnp.int32)
    dst_t = jnp.where(valid, dst[posc], N).astype(jnp.int32)

    def _wave(dst_row, src_row):
        run_start = jnp.searchsorted(dst_row, dst_row, side='left')
        rank = jnp.minimum(jnp.arange(CAP, dtype=jnp.int32) - run_start, RMAX)
        sort_ids = jnp.argsort(rank, stable=True)
        ranks_sorted = rank[sort_ids]
        w = jnp.sum(ranks_sorted[None, :] == jnp.arange(RMAX + 1)[:, None],
                    axis=1)
        wpad = jnp.where(jnp.arange(RMAX + 1) < RMAX,
                         ((w + EB - 1) // EB) * EB, w)
        offs = jnp.concatenate([jnp.zeros((1,), jnp.int32),
                                jnp.cumsum(wpad)[:-1].astype(jnp.int32)])
        in_wave = (jnp.arange(CAP, dtype=jnp.int32)
                   - jnp.searchsorted(ranks_sorted, ranks_sorted, side='left'
                                      ).astype(jnp.int32))
        newpos = offs[ranks_sorted] + in_wave
        out_dst = jnp.full((CAP2,), N, jnp.int32).at[newpos].set(dst_row[sort_ids])
        out_src = jnp.zeros((CAP2,), jnp.int32).at[newpos].set(src_row[sort_ids])
        return out_src, out_dst

    src_main, dst_main = jax.vmap(_wave)(dst_t, src_t)
    src_main = src_main.reshape(NUM_TILES, GROUPS, GLEN, EB)
    dst_main = dst_main.reshape(NUM_TILES, GROUPS, GLEN, EB)
    zeros = jnp.zeros((ROWS_PT, LANE), jnp.float32)

    h_ch = x.reshape(N, 2, LANE).transpose(1, 0, 2)
    nlayers = 4
    for l in range(nlayers):
        cin = h_ch.shape[0]
        agg_ch = _sc_segsum(h_ch, src_main, dst_main, zeros, nchunks=cin)
        eps = params[f'eps_{l}'].reshape(1, 1)
        b1 = params[f'b1_{l}'].reshape(1, HID)
        b2 = params[f'b2_{l}'].reshape(1, HID)
        z_ch, sums = _tc_mlp(h_ch, agg_ch, eps, params[f'W1_{l}'], b1,
                             params[f'W2_{l}'], b2)
        mean = sums[0] / N
        var = sums[1] / N - mean * mean
        inv = lax.rsqrt(var + 1e-5)
        scale = params[f'gamma_{l}'] * inv
        shift = params[f'beta_{l}'] - mean * scale
        h_ch = _tc_norm(z_ch, scale.reshape(HID // LANE, 1, LANE),
                        shift.reshape(HID // LANE, 1, LANE))

    bo = params['b_out'].reshape(1, HID)
    bm = params['b_mu'].reshape(1, -1)
    bl = params['b_lv'].reshape(1, -1)
    return _tc_heads(h_ch, params['W_out'], bo, params['W_mu'], bm,
                     params['W_lv'], bl)


# speedup vs baseline: 1.4184x; 1.4171x over previous
import functools

import jax
import jax.numpy as jnp
from jax import lax
from jax.experimental import pallas as pl
from jax.experimental.pallas import tpu as pltpu
from jax.experimental.pallas import tpu_sc as plsc

N = 10000
E = 160000
HID = 512
LANE = 128

NUM_SC = 2
NUM_TILES = 16
EPT = E // NUM_TILES
EB = 128
SLACK = 184
CAP = EPT + 2 * SLACK
RMAX = 24
CAP2 = CAP + EB * RMAX
NB = CAP2 // EB
GLEN = 15
GROUPS = NB // GLEN
NJUNK = 8
ROWS_PT = 632
ROWS_LAST = N + NJUNK - (NUM_TILES - 1) * ROWS_PT


@functools.partial(jax.jit, static_argnames=("nchunks",))
def _sc_segsum(h_ch, src_main, dst_main, zeros, *, nchunks):
    cpc = nchunks // NUM_SC
    mesh = plsc.VectorSubcoreMesh(core_axis_name="c", subcore_axis_name="s",
                                  num_cores=NUM_SC, num_subcores=NUM_TILES)

    @functools.partial(
        pl.kernel,
        out_type=jax.ShapeDtypeStruct((nchunks, N, LANE), jnp.float32),
        mesh=mesh,
        scratch_types=[
            pltpu.VMEM((GLEN, EB), jnp.int32),
            pltpu.VMEM((GLEN, EB), jnp.int32),
            pltpu.VMEM((EB, LANE), jnp.float32),
            pltpu.VMEM((EB, LANE), jnp.float32),
            pltpu.VMEM_SHARED((N + NJUNK, LANE), jnp.float32),
            pltpu.SemaphoreType.DMA,
            pltpu.SemaphoreType.DMA,
        ],
    )
    def seg(h_hbm, srcm_hbm, dstm_hbm, zeros_hbm, out_hbm,
            src_v, dst_v, rows_a, rows_b, acc, sem_a, sem_b):
        sid = lax.axis_index("s")
        cid = lax.axis_index("c")
        rows = (rows_a, rows_b)
        sems = (sem_a, sem_b)

        def run_core(core):
            def _go():
                off = pl.multiple_of(sid * ROWS_PT, 8)
                for j in range(cpc):
                    chunk = core * cpc + j
                    hc = h_hbm.at[chunk]
                    @pl.when(sid < NUM_TILES - 1)
                    def _():
                        pltpu.sync_copy(zeros_hbm, acc.at[pl.ds(off, ROWS_PT)])

                    @pl.when(sid == NUM_TILES - 1)
                    def _():
                        pltpu.sync_copy(zeros_hbm.at[pl.ds(0, ROWS_LAST)],
                                        acc.at[pl.ds((NUM_TILES - 1) * ROWS_PT,
                                                     ROWS_LAST)])
                    plsc.subcore_barrier()

                    def gbody(g, _):
                        pltpu.sync_copy(srcm_hbm.at[sid].at[g], src_v)
                        pltpu.sync_copy(dstm_hbm.at[sid].at[g], dst_v)
                        pltpu.async_copy(hc.at[src_v.at[0]], rows[0], sems[0])
                        for b in range(GLEN):
                            pltpu.make_async_copy(hc.at[src_v.at[b]],
                                                  rows[b % 2], sems[b % 2]).wait()
                            if b + 1 < GLEN:
                                pltpu.async_copy(hc.at[src_v.at[b + 1]],
                                                 rows[(b + 1) % 2],
                                                 sems[(b + 1) % 2])
                            pltpu.sync_copy(rows[b % 2],
                                            acc.at[dst_v.at[b]], add=True)
                        return 0

                    lax.fori_loop(0, GROUPS, gbody, 0)
                    plsc.subcore_barrier()

                    @pl.when(sid < NUM_TILES - 1)
                    def _():
                        pltpu.sync_copy(acc.at[pl.ds(off, ROWS_PT)],
                                        out_hbm.at[chunk].at[pl.ds(off, ROWS_PT)])

                    @pl.when(sid == NUM_TILES - 1)
                    def _():
                        base = (NUM_TILES - 1) * ROWS_PT
                        nlast = N - base
                        pltpu.sync_copy(acc.at[pl.ds(base, nlast)],
                                        out_hbm.at[chunk].at[pl.ds(base, nlast)])
            return _go

        for core in range(NUM_SC):
            pl.when(cid == core)(run_core(core))

    return seg(h_ch, src_main, dst_main, zeros)


BN_ROWS = 400
GRID = N // BN_ROWS


def _mlp_body(h_ref, a_ref, eps_ref, w1_ref, b1_ref, w2_ref, b2_ref,
              z_ref, s_ref, *, cin):
    i = pl.program_id(0)
    epsv = 1.0 + eps_ref[0, 0]
    z = jnp.concatenate([epsv * h_ref[c] + a_ref[c] for c in range(cin)], axis=1)
    z1 = jnp.dot(z, w1_ref[...], preferred_element_type=jnp.float32) + b1_ref[...]
    z1 = jnp.maximum(z1, 0.0)
    z2 = jnp.dot(z1, w2_ref[...], preferred_element_type=jnp.float32) + b2_ref[...]
    for k in range(HID // LANE):
        z_ref[k] = z2[:, k * LANE:(k + 1) * LANE]
    ps = jnp.concatenate([jnp.sum(z2, axis=0, keepdims=True),
                          jnp.sum(z2 * z2, axis=0, keepdims=True)], axis=0)

    @pl.when(i == 0)
    def _():
        s_ref[...] = ps

    @pl.when(i > 0)
    def _():
        s_ref[...] = s_ref[...] + ps


def _tc_mlp(h_ch, agg_ch, eps, w1, b1, w2, b2):
    cin = h_ch.shape[0]
    co = HID // LANE
    return pl.pallas_call(
        functools.partial(_mlp_body, cin=cin),
        grid=(GRID,),
        in_specs=[
            pl.BlockSpec((cin, BN_ROWS, LANE), lambda i: (0, i, 0)),
            pl.BlockSpec((cin, BN_ROWS, LANE), lambda i: (0, i, 0)),
            pl.BlockSpec(memory_space=pltpu.SMEM),
            pl.BlockSpec((cin * LANE, HID), lambda i: (0, 0)),
            pl.BlockSpec((1, HID), lambda i: (0, 0)),
            pl.BlockSpec((HID, HID), lambda i: (0, 0)),
            pl.BlockSpec((1, HID), lambda i: (0, 0)),
        ],
        out_specs=[
            pl.BlockSpec((co, BN_ROWS, LANE), lambda i: (0, i, 0)),
            pl.BlockSpec((2, HID), lambda i: (0, 0)),
        ],
        out_shape=[
            jax.ShapeDtypeStruct((co, N, LANE), jnp.float32),
            jax.ShapeDtypeStruct((2, HID), jnp.float32),
        ],
    )(h_ch, agg_ch, eps, w1, b1, w2, b2)


def _norm_body(z_ref, sc_ref, sh_ref, h_ref):
    for k in range(HID // LANE):
        h_ref[k] = jnp.maximum(z_ref[k] * sc_ref[k] + sh_ref[k], 0.0)


def _tc_norm(z_ch, scale4, shift4):
    co = HID // LANE
    return pl.pallas_call(
        _norm_body,
        grid=(GRID,),
        in_specs=[
            pl.BlockSpec((co, BN_ROWS, LANE), lambda i: (0, i, 0)),
            pl.BlockSpec((co, 1, LANE), lambda i: (0, 0, 0)),
            pl.BlockSpec((co, 1, LANE), lambda i: (0, 0, 0)),
        ],
        out_specs=pl.BlockSpec((co, BN_ROWS, LANE), lambda i: (0, i, 0)),
        out_shape=jax.ShapeDtypeStruct((co, N, LANE), jnp.float32),
    )(z_ch, scale4, shift4)


def _heads_body(h_ref, wo_ref, bo_ref, wm_ref, bm_ref, wl_ref, bl_ref,
                mu_ref, lv_ref):
    h = jnp.concatenate([h_ref[k] for k in range(HID // LANE)], axis=1)
    ne = jnp.dot(h, wo_ref[...], preferred_element_type=jnp.float32) + bo_ref[...]
    mu_ref[...] = jnp.dot(ne, wm_ref[...], preferred_element_type=jnp.float32) + bm_ref[...]
    lv_ref[...] = jnp.dot(ne, wl_ref[...], preferred_element_type=jnp.float32) + bl_ref[...]


def _tc_heads(h_ch, wo, bo, wm, bm, wl, bl):
    co = HID // LANE
    dv = wm.shape[1]
    return pl.pallas_call(
        _heads_body,
        grid=(GRID,),
        in_specs=[
            pl.BlockSpec((co, BN_ROWS, LANE), lambda i: (0, i, 0)),
            pl.BlockSpec((HID, HID), lambda i: (0, 0)),
            pl.BlockSpec((1, HID), lambda i: (0, 0)),
            pl.BlockSpec((HID, dv), lambda i: (0, 0)),
            pl.BlockSpec((1, dv), lambda i: (0, 0)),
            pl.BlockSpec((HID, dv), lambda i: (0, 0)),
            pl.BlockSpec((1, dv), lambda i: (0, 0)),
        ],
        out_specs=[
            pl.BlockSpec((BN_ROWS, dv), lambda i: (i, 0)),
            pl.BlockSpec((BN_ROWS, dv), lambda i: (i, 0)),
        ],
        out_shape=[
            jax.ShapeDtypeStruct((N, dv), jnp.float32),
            jax.ShapeDtypeStruct((N, dv), jnp.float32),
        ],
    )(h_ch, wo, bo, wm, bm, wl, bl)


def kernel(x, edge_index, params):
    order = jnp.argsort(edge_index[1], stable=True)
    src = edge_index[0][order]
    dst = edge_index[1][order]
    targets = jnp.arange(1, NUM_TILES) * EPT
    raw = jnp.searchsorted(dst, dst[targets], side='left')
    bnd = jnp.clip(raw, targets - SLACK, targets + SLACK)
    bnd = jnp.concatenate([jnp.zeros((1,), bnd.dtype), bnd,
                           jnp.full((1,), E, bnd.dtype)])
    pos = bnd[:NUM_TILES, None] + jnp.arange(CAP)[None, :]
    valid = pos < bnd[1:, None]
    posc = jnp.minimum(pos, E - 1)
    src_t = jnp.where(valid, src[posc], 0).astype(jnp.int32)
    dst_t = jnp.where(valid, dst[posc], N).astype(jnp.int32)

    def _wave(dst_row, src_row):
        run_start = jnp.searchsorted(dst_row, dst_row, side='left')
        rank = jnp.minimum(jnp.arange(CAP, dtype=jnp.int32) - run_start, RMAX)
        sort_ids = jnp.argsort(rank, stable=True)
        ranks_sorted = rank[sort_ids]
        w = jnp.sum(ranks_sorted[None, :] == jnp.arange(RMAX + 1)[:, None],
                    axis=1)
        wpad = jnp.where(jnp.arange(RMAX + 1) < RMAX,
                         ((w + EB - 1) // EB) * EB, w)
        offs = jnp.concatenate([jnp.zeros((1,), jnp.int32),
                                jnp.cumsum(wpad)[:-1].astype(jnp.int32)])
        in_wave = (jnp.arange(CAP, dtype=jnp.int32)
                   - jnp.searchsorted(ranks_sorted, ranks_sorted, side='left'
                                      ).astype(jnp.int32))
        newpos = offs[ranks_sorted] + in_wave
        out_dst = jnp.full((CAP2,), N, jnp.int32).at[newpos].set(dst_row[sort_ids])
        out_src = jnp.zeros((CAP2,), jnp.int32).at[newpos].set(src_row[sort_ids])
        return out_src, out_dst

    src_main, dst_main = jax.vmap(_wave)(dst_t, src_t)
    src_main = src_main.reshape(NUM_TILES, GROUPS, GLEN, EB)
    dst_main = dst_main.reshape(NUM_TILES, GROUPS, GLEN, EB)
    zeros = jnp.zeros((ROWS_PT, LANE), jnp.float32)

    h_ch = x.reshape(N, 2, LANE).transpose(1, 0, 2)
    nlayers = 4
    for l in range(nlayers):
        cin = h_ch.shape[0]
        agg_ch = _sc_segsum(h_ch, src_main, dst_main, zeros, nchunks=cin)
        eps = params[f'eps_{l}'].reshape(1, 1)
        b1 = params[f'b1_{l}'].reshape(1, HID)
        b2 = params[f'b2_{l}'].reshape(1, HID)
        z_ch, sums = _tc_mlp(h_ch, agg_ch, eps, params[f'W1_{l}'], b1,
                             params[f'W2_{l}'], b2)
        mean = sums[0] / N
        var = sums[1] / N - mean * mean
        inv = lax.rsqrt(var + 1e-5)
        scale = params[f'gamma_{l}'] * inv
        shift = params[f'beta_{l}'] - mean * scale
        h_ch = _tc_norm(z_ch, scale.reshape(HID // LANE, 1, LANE),
                        shift.reshape(HID // LANE, 1, LANE))

    bo = params['b_out'].reshape(1, HID)
    bm = params['b_mu'].reshape(1, -1)
    bl = params['b_lv'].reshape(1, -1)
    return _tc_heads(h_ch, params['W_out'], bo, params['W_mu'], bm,
                     params['W_lv'], bl)


# speedup vs baseline: 1.4186x; 1.0002x over previous
import functools

import jax
import jax.numpy as jnp
from jax import lax
from jax.experimental import pallas as pl
from jax.experimental.pallas import tpu as pltpu
from jax.experimental.pallas import tpu_sc as plsc

N = 10000
E = 160000
HID = 512
LANE = 128

NUM_SC = 2
NUM_TILES = 16
EPT = E // NUM_TILES
EB = 128
SLACK = 184
CAP = EPT + 2 * SLACK
RMAX = 24
CAP2 = CAP + EB * RMAX
NB = CAP2 // EB
GLEN = 35
GROUPS = NB // GLEN
NJUNK = 8
ROWS_PT = 632
ROWS_LAST = N + NJUNK - (NUM_TILES - 1) * ROWS_PT


@functools.partial(jax.jit, static_argnames=("nchunks",))
def _sc_segsum(h_ch, src_main, dst_main, zeros, *, nchunks):
    cpc = nchunks // NUM_SC
    mesh = plsc.VectorSubcoreMesh(core_axis_name="c", subcore_axis_name="s",
                                  num_cores=NUM_SC, num_subcores=NUM_TILES)

    @functools.partial(
        pl.kernel,
        out_type=jax.ShapeDtypeStruct((nchunks, N, LANE), jnp.float32),
        mesh=mesh,
        scratch_types=[
            pltpu.VMEM((GLEN, EB), jnp.int32),
            pltpu.VMEM((GLEN, EB), jnp.int32),
            pltpu.VMEM((EB, LANE), jnp.float32),
            pltpu.VMEM((EB, LANE), jnp.float32),
            pltpu.VMEM_SHARED((N + NJUNK, LANE), jnp.float32),
            pltpu.SemaphoreType.DMA,
            pltpu.SemaphoreType.DMA,
        ],
    )
    def seg(h_hbm, srcm_hbm, dstm_hbm, zeros_hbm, out_hbm,
            src_v, dst_v, rows_a, rows_b, acc, sem_a, sem_b):
        sid = lax.axis_index("s")
        cid = lax.axis_index("c")
        rows = (rows_a, rows_b)
        sems = (sem_a, sem_b)

        def run_core(core):
            def _go():
                off = pl.multiple_of(sid * ROWS_PT, 8)
                for j in range(cpc):
                    chunk = core * cpc + j
                    hc = h_hbm.at[chunk]
                    @pl.when(sid < NUM_TILES - 1)
                    def _():
                        pltpu.sync_copy(zeros_hbm, acc.at[pl.ds(off, ROWS_PT)])

                    @pl.when(sid == NUM_TILES - 1)
                    def _():
                        pltpu.sync_copy(zeros_hbm.at[pl.ds(0, ROWS_LAST)],
                                        acc.at[pl.ds((NUM_TILES - 1) * ROWS_PT,
                                                     ROWS_LAST)])
                    plsc.subcore_barrier()

                    def gbody(g, _):
                        pltpu.sync_copy(srcm_hbm.at[sid].at[g], src_v)
                        pltpu.sync_copy(dstm_hbm.at[sid].at[g], dst_v)
                        pltpu.async_copy(hc.at[src_v.at[0]], rows[0], sems[0])
                        for b in range(GLEN):
                            pltpu.make_async_copy(hc.at[src_v.at[b]],
                                                  rows[b % 2], sems[b % 2]).wait()
                            if b + 1 < GLEN:
                                pltpu.async_copy(hc.at[src_v.at[b + 1]],
                                                 rows[(b + 1) % 2],
                                                 sems[(b + 1) % 2])
                            pltpu.sync_copy(rows[b % 2],
                                            acc.at[dst_v.at[b]], add=True)
                        return 0

                    lax.fori_loop(0, GROUPS, gbody, 0)
                    plsc.subcore_barrier()

                    @pl.when(sid < NUM_TILES - 1)
                    def _():
                        pltpu.sync_copy(acc.at[pl.ds(off, ROWS_PT)],
                                        out_hbm.at[chunk].at[pl.ds(off, ROWS_PT)])

                    @pl.when(sid == NUM_TILES - 1)
                    def _():
                        base = (NUM_TILES - 1) * ROWS_PT
                        nlast = N - base
                        pltpu.sync_copy(acc.at[pl.ds(base, nlast)],
                                        out_hbm.at[chunk].at[pl.ds(base, nlast)])
            return _go

        for core in range(NUM_SC):
            pl.when(cid == core)(run_core(core))

    return seg(h_ch, src_main, dst_main, zeros)


BN_ROWS = 400
GRID = N // BN_ROWS


def _mlp_body(h_ref, a_ref, eps_ref, w1_ref, b1_ref, w2_ref, b2_ref,
              z_ref, s_ref, *, cin):
    i = pl.program_id(0)
    epsv = 1.0 + eps_ref[0, 0]
    z = jnp.concatenate([epsv * h_ref[c] + a_ref[c] for c in range(cin)], axis=1)
    z1 = jnp.dot(z, w1_ref[...], preferred_element_type=jnp.float32) + b1_ref[...]
    z1 = jnp.maximum(z1, 0.0)
    z2 = jnp.dot(z1, w2_ref[...], preferred_element_type=jnp.float32) + b2_ref[...]
    for k in range(HID // LANE):
        z_ref[k] = z2[:, k * LANE:(k + 1) * LANE]
    ps = jnp.concatenate([jnp.sum(z2, axis=0, keepdims=True),
                          jnp.sum(z2 * z2, axis=0, keepdims=True)], axis=0)

    @pl.when(i == 0)
    def _():
        s_ref[...] = ps

    @pl.when(i > 0)
    def _():
        s_ref[...] = s_ref[...] + ps


def _tc_mlp(h_ch, agg_ch, eps, w1, b1, w2, b2):
    cin = h_ch.shape[0]
    co = HID // LANE
    return pl.pallas_call(
        functools.partial(_mlp_body, cin=cin),
        grid=(GRID,),
        in_specs=[
            pl.BlockSpec((cin, BN_ROWS, LANE), lambda i: (0, i, 0)),
            pl.BlockSpec((cin, BN_ROWS, LANE), lambda i: (0, i, 0)),
            pl.BlockSpec(memory_space=pltpu.SMEM),
            pl.BlockSpec((cin * LANE, HID), lambda i: (0, 0)),
            pl.BlockSpec((1, HID), lambda i: (0, 0)),
            pl.BlockSpec((HID, HID), lambda i: (0, 0)),
            pl.BlockSpec((1, HID), lambda i: (0, 0)),
        ],
        out_specs=[
            pl.BlockSpec((co, BN_ROWS, LANE), lambda i: (0, i, 0)),
            pl.BlockSpec((2, HID), lambda i: (0, 0)),
        ],
        out_shape=[
            jax.ShapeDtypeStruct((co, N, LANE), jnp.float32),
            jax.ShapeDtypeStruct((2, HID), jnp.float32),
        ],
    )(h_ch, agg_ch, eps, w1, b1, w2, b2)


def _norm_body(z_ref, sc_ref, sh_ref, h_ref):
    for k in range(HID // LANE):
        h_ref[k] = jnp.maximum(z_ref[k] * sc_ref[k] + sh_ref[k], 0.0)


def _tc_norm(z_ch, scale4, shift4):
    co = HID // LANE
    return pl.pallas_call(
        _norm_body,
        grid=(GRID,),
        in_specs=[
            pl.BlockSpec((co, BN_ROWS, LANE), lambda i: (0, i, 0)),
            pl.BlockSpec((co, 1, LANE), lambda i: (0, 0, 0)),
            pl.BlockSpec((co, 1, LANE), lambda i: (0, 0, 0)),
        ],
        out_specs=pl.BlockSpec((co, BN_ROWS, LANE), lambda i: (0, i, 0)),
        out_shape=jax.ShapeDtypeStruct((co, N, LANE), jnp.float32),
    )(z_ch, scale4, shift4)


def _heads_body(h_ref, wo_ref, bo_ref, wm_ref, bm_ref, wl_ref, bl_ref,
                mu_ref, lv_ref):
    h = jnp.concatenate([h_ref[k] for k in range(HID // LANE)], axis=1)
    ne = jnp.dot(h, wo_ref[...], preferred_element_type=jnp.float32) + bo_ref[...]
    mu_ref[...] = jnp.dot(ne, wm_ref[...], preferred_element_type=jnp.float32) + bm_ref[...]
    lv_ref[...] = jnp.dot(ne, wl_ref[...], preferred_element_type=jnp.float32) + bl_ref[...]


def _tc_heads(h_ch, wo, bo, wm, bm, wl, bl):
    co = HID // LANE
    dv = wm.shape[1]
    return pl.pallas_call(
        _heads_body,
        grid=(GRID,),
        in_specs=[
            pl.BlockSpec((co, BN_ROWS, LANE), lambda i: (0, i, 0)),
            pl.BlockSpec((HID, HID), lambda i: (0, 0)),
            pl.BlockSpec((1, HID), lambda i: (0, 0)),
            pl.BlockSpec((HID, dv), lambda i: (0, 0)),
            pl.BlockSpec((1, dv), lambda i: (0, 0)),
            pl.BlockSpec((HID, dv), lambda i: (0, 0)),
            pl.BlockSpec((1, dv), lambda i: (0, 0)),
        ],
        out_specs=[
            pl.BlockSpec((BN_ROWS, dv), lambda i: (i, 0)),
            pl.BlockSpec((BN_ROWS, dv), lambda i: (i, 0)),
        ],
        out_shape=[
            jax.ShapeDtypeStruct((N, dv), jnp.float32),
            jax.ShapeDtypeStruct((N, dv), jnp.float32),
        ],
    )(h_ch, wo, bo, wm, bm, wl, bl)


def kernel(x, edge_index, params):
    order = jnp.argsort(edge_index[1], stable=True)
    src = edge_index[0][order]
    dst = edge_index[1][order]
    targets = jnp.arange(1, NUM_TILES) * EPT
    raw = jnp.searchsorted(dst, dst[targets], side='left')
    bnd = jnp.clip(raw, targets - SLACK, targets + SLACK)
    bnd = jnp.concatenate([jnp.zeros((1,), bnd.dtype), bnd,
                           jnp.full((1,), E, bnd.dtype)])
    pos = bnd[:NUM_TILES, None] + jnp.arange(CAP)[None, :]
    valid = pos < bnd[1:, None]
    posc = jnp.minimum(pos, E - 1)
    src_t = jnp.where(valid, src[posc], 0).astype(jnp.int32)
    dst_t = jnp.where(valid, dst[posc], N).astype(jnp.int32)

    def _wave(dst_row, src_row):
        run_start = jnp.searchsorted(dst_row, dst_row, side='left')
        rank = jnp.minimum(jnp.arange(CAP, dtype=jnp.int32) - run_start, RMAX)
        sort_ids = jnp.argsort(rank, stable=True)
        ranks_sorted = rank[sort_ids]
        w = jnp.sum(ranks_sorted[None, :] == jnp.arange(RMAX + 1)[:, None],
                    axis=1)
        wpad = jnp.where(jnp.arange(RMAX + 1) < RMAX,
                         ((w + EB - 1) // EB) * EB, w)
        offs = jnp.concatenate([jnp.zeros((1,), jnp.int32),
                                jnp.cumsum(wpad)[:-1].astype(jnp.int32)])
        in_wave = (jnp.arange(CAP, dtype=jnp.int32)
                   - jnp.searchsorted(ranks_sorted, ranks_sorted, side='left'
                                      ).astype(jnp.int32))
        newpos = offs[ranks_sorted] + in_wave
        out_dst = jnp.full((CAP2,), N, jnp.int32).at[newpos].set(dst_row[sort_ids])
        out_src = jnp.zeros((CAP2,), jnp.int32).at[newpos].set(src_row[sort_ids])
        return out_src, out_dst

    src_main, dst_main = jax.vmap(_wave)(dst_t, src_t)
    src_main = src_main.reshape(NUM_TILES, GROUPS, GLEN, EB)
    dst_main = dst_main.reshape(NUM_TILES, GROUPS, GLEN, EB)
    zeros = jnp.zeros((ROWS_PT, LANE), jnp.float32)

    h_ch = x.reshape(N, 2, LANE).transpose(1, 0, 2)
    nlayers = 4
    for l in range(nlayers):
        cin = h_ch.shape[0]
        agg_ch = _sc_segsum(h_ch, src_main, dst_main, zeros, nchunks=cin)
        eps = params[f'eps_{l}'].reshape(1, 1)
        b1 = params[f'b1_{l}'].reshape(1, HID)
        b2 = params[f'b2_{l}'].reshape(1, HID)
        z_ch, sums = _tc_mlp(h_ch, agg_ch, eps, params[f'W1_{l}'], b1,
                             params[f'W2_{l}'], b2)
        mean = sums[0] / N
        var = sums[1] / N - mean * mean
        inv = lax.rsqrt(var + 1e-5)
        scale = params[f'gamma_{l}'] * inv
        shift = params[f'beta_{l}'] - mean * scale
        h_ch = _tc_norm(z_ch, scale.reshape(HID // LANE, 1, LANE),
                        shift.reshape(HID // LANE, 1, LANE))

    bo = params['b_out'].reshape(1, HID)
    bm = params['b_mu'].reshape(1, -1)
    bl = params['b_lv'].reshape(1, -1)
    return _tc_heads(h_ch, params['W_out'], bo, params['W_mu'], bm,
                     params['W_lv'], bl)
